# baseline (speedup 1.0000x reference)
import functools

import jax
import jax.numpy as jnp
from jax import lax
from jax.experimental import pallas as pl
from jax.experimental.pallas import tpu as pltpu
from jax.experimental.pallas import tpu_sc as plsc

_NC = 2
_NS = 16
_CHUNK = 64


def _make_sc_copy(batch, seq_len, d_model, dtype):
    nw = _NC * _NS
    rows_per_w = seq_len // nw
    n_chunks = rows_per_w // _CHUNK
    mesh = plsc.VectorSubcoreMesh(core_axis_name="c", subcore_axis_name="s")

    @functools.partial(
        pl.kernel,
        mesh=mesh,
        out_type=jax.ShapeDtypeStruct((batch, seq_len, d_model), dtype),
        scratch_types=[
            pltpu.VMEM((_CHUNK, d_model), dtype),
            pltpu.VMEM((_CHUNK, d_model), dtype),
            pltpu.SemaphoreType.DMA,
            pltpu.SemaphoreType.DMA,
        ],
    )
    def sc_copy(emb_hbm, out_hbm, buf0, buf1, in_sem, out_sem):
        wid = lax.axis_index("s") * _NC + lax.axis_index("c")
        base = wid * rows_per_w
        bufs = (buf0, buf1)
        ins = [None] * n_chunks
        outs = [None] * n_chunks
        for c in range(min(2, n_chunks)):
            ins[c] = pltpu.async_copy(
                emb_hbm.at[pl.ds(base + c * _CHUNK, _CHUNK)], bufs[c], in_sem)
        for c in range(n_chunks):
            ins[c].wait()
            row0 = base + c * _CHUNK
            outs[c] = [
                pltpu.async_copy(bufs[c % 2],
                                 out_hbm.at[b, pl.ds(row0, _CHUNK)], out_sem)
                for b in range(batch)
            ]
            if c + 2 < n_chunks:
                for h in outs[c]:
                    h.wait()
                ins[c + 2] = pltpu.async_copy(
                    emb_hbm.at[pl.ds(base + (c + 2) * _CHUNK, _CHUNK)],
                    bufs[c % 2], in_sem)
        for c in range(max(n_chunks - 2, 0), n_chunks):
            for h in outs[c]:
                h.wait()

    return sc_copy


def kernel(x, pos_emb):
    batch, seq_len = x.shape
    max_len, d_model = pos_emb.shape
    fn = _make_sc_copy(batch, seq_len, d_model, pos_emb.dtype)
    return fn(pos_emb)
